# baseline (speedup 1.0000x reference)
import functools

import jax
import jax.numpy as jnp
from jax import lax
from jax.experimental import pallas as pl
from jax.experimental.pallas import tpu as pltpu
from jax.experimental.pallas import tpu_sc as plsc

_B = 16384
_D = 64


def kernel(user_table, item_table, user_idx, item_idx):
    info = plsc.get_sparse_core_info()
    nw = info.num_cores * info.num_subcores
    bpw = _B // nw

    mesh = plsc.VectorSubcoreMesh(core_axis_name="c", subcore_axis_name="s")

    @functools.partial(
        pl.kernel,
        mesh=mesh,
        out_type=jax.ShapeDtypeStruct((2 * _B * _D,), jnp.float32),
        scratch_types=[
            pltpu.VMEM((bpw,), jnp.int32),
            pltpu.VMEM((bpw,), jnp.int32),
            pltpu.VMEM((bpw * _D,), jnp.float32),
            pltpu.VMEM((bpw * _D,), jnp.float32),
            pltpu.SemaphoreType.DMA,
            pltpu.SemaphoreType.DMA,
        ],
        compiler_params=pltpu.CompilerParams(use_tc_tiling_on_sc=False),
    )
    def _emb(ut, it, ui, ii, out, uidx_v, iidx_v, urows_v, irows_v,
             usem, isem):
        wid = lax.axis_index("s") * info.num_cores + lax.axis_index("c")
        base = wid * bpw
        pltpu.sync_copy(ui.at[pl.ds(base, bpw)], uidx_v)
        pltpu.sync_copy(ii.at[pl.ds(base, bpw)], iidx_v)

        def enqueue(tbl, idx_v, rows_v, sem):
            def body(g, carry):
                off = idx_v[pl.ds(g * 16, 16)] * _D
                for k in range(16):
                    pltpu.async_copy(
                        tbl.at[pl.ds(pl.multiple_of(off[k], _D), _D)],
                        rows_v.at[pl.ds((g * 16 + k) * _D, _D)],
                        sem)
                return carry
            lax.fori_loop(0, bpw // 16, body, 0)

        enqueue(ut, uidx_v, urows_v, usem)
        enqueue(it, iidx_v, irows_v, isem)
        pltpu.make_async_copy(ut.at[pl.ds(0, bpw * _D)], urows_v, usem).wait()
        pltpu.sync_copy(urows_v, out.at[pl.ds(base * _D, bpw * _D)])
        pltpu.make_async_copy(it.at[pl.ds(0, bpw * _D)], irows_v, isem).wait()
        pltpu.sync_copy(irows_v,
                        out.at[pl.ds((_B + base) * _D, bpw * _D)])

    out = _emb(user_table.reshape(-1), item_table.reshape(-1),
               user_idx.astype(jnp.int32), item_idx.astype(jnp.int32))
    return out.reshape(2, _B, _D)


# speedup vs baseline: 1.3418x; 1.3418x over previous
import functools

import jax
import jax.numpy as jnp
from jax import lax
from jax.experimental import pallas as pl
from jax.experimental.pallas import tpu as pltpu
from jax.experimental.pallas import tpu_sc as plsc

_B = 16384
_D = 64
_CHUNK = 256


def _make_gather():
    info = plsc.get_sparse_core_info()
    nw = info.num_cores * info.num_subcores
    bpw = _B // nw

    mesh = plsc.VectorSubcoreMesh(core_axis_name="c", subcore_axis_name="s")

    @functools.partial(
        pl.kernel,
        mesh=mesh,
        out_type=jax.ShapeDtypeStruct((_B, _D), jnp.float32),
        scratch_types=[
            pltpu.VMEM((bpw,), jnp.int32),
            pltpu.VMEM((_CHUNK, _D), jnp.float32),
            pltpu.VMEM((_CHUNK, _D), jnp.float32),
            pltpu.SemaphoreType.DMA,
            pltpu.SemaphoreType.DMA,
        ],
        compiler_params=pltpu.CompilerParams(use_tc_tiling_on_sc=True),
    )
    def _gather(tbl, idx, out, idx_v, buf_a, buf_b, sem_a, sem_b):
        wid = lax.axis_index("s") * info.num_cores + lax.axis_index("c")
        base = wid * bpw
        pltpu.sync_copy(idx.at[pl.ds(base, bpw)], idx_v)

        def enqueue(buf, sem, c):
            def body(g, carry):
                vec = idx_v[pl.ds(c * _CHUNK + g * 16, 16)]
                for k in range(16):
                    pltpu.async_copy(tbl.at[vec[k]], buf.at[g * 16 + k], sem)
                return carry
            lax.fori_loop(0, _CHUNK // 16, body, 0)

        enqueue(buf_a, sem_a, 0)
        enqueue(buf_b, sem_b, 1)
        pltpu.make_async_copy(tbl.at[pl.ds(0, _CHUNK)], buf_a, sem_a).wait()
        pltpu.sync_copy(buf_a, out.at[pl.ds(base, _CHUNK)])
        pltpu.make_async_copy(tbl.at[pl.ds(0, _CHUNK)], buf_b, sem_b).wait()
        pltpu.sync_copy(buf_b, out.at[pl.ds(base + _CHUNK, _CHUNK)])

    return _gather


def kernel(user_table, item_table, user_idx, item_idx):
    gather = _make_gather()
    user_x = gather(user_table, user_idx.astype(jnp.int32))
    item_x = gather(item_table, item_idx.astype(jnp.int32))
    return jnp.stack([user_x, item_x], axis=0)
